# baseline (speedup 1.0000x reference)
import functools

import jax
import jax.numpy as jnp
from jax import lax
from jax.experimental import pallas as pl
from jax.experimental.pallas import tpu as pltpu
from jax.experimental.pallas import tpu_sc as plsc

_L = 16
_PAD = 128


def _make_sc_kernel(B, L):
    nblk = L // _L
    rows_per_core = B
    mesh = plsc.VectorSubcoreMesh(core_axis_name="c", subcore_axis_name="s",
                                  num_cores=1)

    @functools.partial(
        pl.kernel,
        mesh=mesh,
        compiler_params=pltpu.CompilerParams(needs_layout_passes=False),
        out_type=(
            jax.ShapeDtypeStruct((B, L), jnp.int32),
            jax.ShapeDtypeStruct((B, L), jnp.int32),
            jax.ShapeDtypeStruct((B, L), jnp.int32),
            jax.ShapeDtypeStruct((B, _L), jnp.int32),
        ),
        scratch_types=(
            pltpu.VMEM((L + 2 * _PAD,), jnp.int32),
            pltpu.VMEM((L,), jnp.int32),
            pltpu.VMEM((L,), jnp.int32),
            pltpu.VMEM((L,), jnp.int32),
            pltpu.VMEM((_L,), jnp.int32),
        ),
    )
    def tok_kernel(chars_hbm, tid_hbm, st_hbm, en_hbm, nt_hbm,
                   padded, tid_out, st_out, en_out, nt_vec):
        cid = lax.axis_index("c")
        sid = lax.axis_index("s")
        zero = jnp.zeros((_L,), jnp.int32)

        del cid
        if True:
            row = sid
            padded[pl.ds(_PAD - _L, _L)] = zero
            padded[pl.ds(_PAD + L, _L)] = zero
            pltpu.sync_copy(chars_hbm.at[row], padded.at[pl.ds(_PAD, L)])

            unroll = 4

            def body(g, cr):
                neg1 = jnp.full((_L,), -1, jnp.int32)
                one = jnp.full((_L,), 1, jnp.int32)
                gbase = g * (_L * unroll)
                blocks = []
                for u in range(unroll):
                    base = gbase + u * _L
                    st_out[pl.ds(base, _L)] = neg1
                    en_out[pl.ds(base, _L)] = neg1
                    c = padded[pl.ds(base + _PAD, _L)]
                    p = padded[pl.ds(base + _PAD - 1, _L)]
                    n = padded[pl.ds(base + _PAD + 1, _L)]
                    is_tok = c != 0
                    start_m = is_tok & (p == 0)
                    end_m = is_tok & (n == 0)
                    cum = plsc.cumsum(jnp.where(start_m, one, zero))
                    pc = plsc.all_reduce_population_count(start_m)
                    blocks.append((base, is_tok, start_m, end_m, cum, pc))
                for base, is_tok, start_m, end_m, cum, pc in blocks:
                    tid = cr + cum - 1
                    tid_out[pl.ds(base, _L)] = jnp.where(is_tok, tid, neg1)
                    idx = jnp.maximum(tid, 0)
                    pos = lax.iota(jnp.int32, _L) + base
                    plsc.store_scatter(st_out, [idx], pos, mask=start_m)
                    plsc.store_scatter(en_out, [idx], pos + 1, mask=end_m)
                    cr = cr + pc
                return cr

            carry = lax.fori_loop(0, nblk // unroll, body, zero)
            nt_vec[...] = carry
            pltpu.sync_copy(tid_out, tid_hbm.at[row])
            pltpu.sync_copy(st_out, st_hbm.at[row])
            pltpu.sync_copy(en_out, en_hbm.at[row])
            pltpu.sync_copy(nt_vec, nt_hbm.at[row])

    return tok_kernel


def kernel(chars):
    B, L = chars.shape
    tid, st, en, nt_stage = _make_sc_kernel(B, L)(chars)
    return (tid, st, en, nt_stage[:, 0])


# speedup vs baseline: 1.0229x; 1.0229x over previous
import functools

import jax
import jax.numpy as jnp
from jax import lax
from jax.experimental import pallas as pl
from jax.experimental.pallas import tpu as pltpu
from jax.experimental.pallas import tpu_sc as plsc

_L = 16
_PAD = 128


def _make_sc_kernel(B, L):
    nblk = L // _L
    rows_per_core = B
    mesh = plsc.VectorSubcoreMesh(core_axis_name="c", subcore_axis_name="s",
                                  num_cores=1)

    @functools.partial(
        pl.kernel,
        mesh=mesh,
        compiler_params=pltpu.CompilerParams(needs_layout_passes=False),
        out_type=(
            jax.ShapeDtypeStruct((B, L), jnp.int32),
            jax.ShapeDtypeStruct((B, L), jnp.int32),
            jax.ShapeDtypeStruct((B, L), jnp.int32),
            jax.ShapeDtypeStruct((B,), jnp.int32),
            jax.ShapeDtypeStruct((B, _L), jnp.int32),
        ),
        scratch_types=(
            pltpu.VMEM((L + 2 * _PAD,), jnp.int32),
            pltpu.VMEM((L,), jnp.int32),
            pltpu.VMEM((L,), jnp.int32),
            pltpu.VMEM((L,), jnp.int32),
            pltpu.VMEM((_L,), jnp.int32),
            pltpu.VMEM((B, _L), jnp.int32),
        ),
    )
    def tok_kernel(chars_hbm, tid_hbm, st_hbm, en_hbm, nt_hbm, xchg_hbm,
                   padded, tid_out, st_out, en_out, nt_vec, agg):
        cid = lax.axis_index("c")
        sid = lax.axis_index("s")
        zero = jnp.zeros((_L,), jnp.int32)

        del cid
        if True:
            row = sid
            padded[pl.ds(_PAD - _L, _L)] = zero
            padded[pl.ds(_PAD + L, _L)] = zero
            pltpu.sync_copy(chars_hbm.at[row], padded.at[pl.ds(_PAD, L)])

            unroll = 4

            def body(g, cr):
                neg1 = jnp.full((_L,), -1, jnp.int32)
                one = jnp.full((_L,), 1, jnp.int32)
                gbase = g * (_L * unroll)
                blocks = []
                for u in range(unroll):
                    base = gbase + u * _L
                    st_out[pl.ds(base, _L)] = neg1
                    en_out[pl.ds(base, _L)] = neg1
                    c = padded[pl.ds(base + _PAD, _L)]
                    p = padded[pl.ds(base + _PAD - 1, _L)]
                    n = padded[pl.ds(base + _PAD + 1, _L)]
                    is_tok = c != 0
                    start_m = is_tok & (p == 0)
                    end_m = is_tok & (n == 0)
                    cum = plsc.cumsum(jnp.where(start_m, one, zero))
                    pc = plsc.all_reduce_population_count(start_m)
                    blocks.append((base, is_tok, start_m, end_m, cum, pc))
                for base, is_tok, start_m, end_m, cum, pc in blocks:
                    tid = cr + cum - 1
                    tid_out[pl.ds(base, _L)] = jnp.where(is_tok, tid, neg1)
                    idx = jnp.maximum(tid, 0)
                    pos = lax.iota(jnp.int32, _L) + base
                    plsc.store_scatter(st_out, [idx], pos, mask=start_m)
                    plsc.store_scatter(en_out, [idx], pos + 1, mask=end_m)
                    cr = cr + pc
                return cr

            carry = lax.fori_loop(0, nblk // unroll, body, zero)
            nt_vec[...] = carry
            pltpu.sync_copy(tid_out, tid_hbm.at[row])
            pltpu.sync_copy(st_out, st_hbm.at[row])
            pltpu.sync_copy(en_out, en_hbm.at[row])
            pltpu.sync_copy(nt_vec, xchg_hbm.at[row])

        plsc.subcore_barrier()

        @pl.when(sid == 0)
        def _write_counts():
            pltpu.sync_copy(xchg_hbm, agg)
            iot = lax.iota(jnp.int32, _L)
            nt_vec[...] = plsc.load_gather(agg, [iot, iot])
            pltpu.sync_copy(nt_vec, nt_hbm)

    return tok_kernel


def kernel(chars):
    B, L = chars.shape
    tid, st, en, nt, _unused_xchg = _make_sc_kernel(B, L)(chars)
    return (tid, st, en, nt)


# speedup vs baseline: 1.0275x; 1.0045x over previous
import functools

import jax
import jax.numpy as jnp
from jax import lax
from jax.experimental import pallas as pl
from jax.experimental.pallas import tpu as pltpu
from jax.experimental.pallas import tpu_sc as plsc

_L = 16
_PAD = 128
_UNROLL = 4


def _make_sc_kernel(B, L):
    nblk = L // _L
    mesh = plsc.VectorSubcoreMesh(core_axis_name="c", subcore_axis_name="s",
                                  num_cores=1)

    @functools.partial(
        pl.kernel,
        mesh=mesh,
        compiler_params=pltpu.CompilerParams(needs_layout_passes=False),
        out_type=(
            jax.ShapeDtypeStruct((B, L), jnp.int32),
            jax.ShapeDtypeStruct((B, L), jnp.int32),
            jax.ShapeDtypeStruct((B, L), jnp.int32),
            jax.ShapeDtypeStruct((B,), jnp.int32),
            jax.ShapeDtypeStruct((B, _L), jnp.int32),
        ),
        scratch_types=(
            pltpu.VMEM((L + 2 * _PAD,), jnp.int32),
            pltpu.VMEM((L,), jnp.int32),
            pltpu.VMEM((L,), jnp.int32),
            pltpu.VMEM((L,), jnp.int32),
            pltpu.VMEM((_L,), jnp.int32),
            pltpu.VMEM((B, _L), jnp.int32),
        ),
    )
    def tok_kernel(chars_hbm, tid_hbm, st_hbm, en_hbm, nt_hbm, xchg_hbm,
                   padded, tid_out, st_out, en_out, nt_vec, agg):
        row = lax.axis_index("s")
        zero = jnp.zeros((_L,), jnp.int32)

        padded[pl.ds(_PAD - _L, _L)] = zero
        padded[pl.ds(_PAD + L, _L)] = zero
        pltpu.sync_copy(chars_hbm.at[row], padded.at[pl.ds(_PAD, L)])

        def body(g, cr):
            neg1 = jnp.full((_L,), -1, jnp.int32)
            one = jnp.full((_L,), 1, jnp.int32)
            gbase = g * (_L * _UNROLL)
            blocks = []
            for u in range(_UNROLL):
                base = gbase + u * _L
                st_out[pl.ds(base, _L)] = neg1
                en_out[pl.ds(base, _L)] = neg1
                c = padded[pl.ds(base + _PAD, _L)]
                p = padded[pl.ds(base + _PAD - 1, _L)]
                n = padded[pl.ds(base + _PAD + 1, _L)]
                is_tok = c != 0
                start_m = is_tok & (p == 0)
                end_m = is_tok & (n == 0)
                cum = plsc.cumsum(jnp.where(start_m, one, zero))
                pc = plsc.all_reduce_population_count(start_m)
                blocks.append((base, is_tok, start_m, end_m, cum, pc))
            for base, is_tok, start_m, end_m, cum, pc in blocks:
                tid = cr + cum - 1
                tid_out[pl.ds(base, _L)] = jnp.where(is_tok, tid, neg1)
                idx = jnp.maximum(tid, 0)
                pos = lax.iota(jnp.int32, _L) + base
                plsc.store_scatter(st_out, [idx], pos, mask=start_m)
                plsc.store_scatter(en_out, [idx], pos + 1, mask=end_m)
                cr = cr + pc
            return cr

        carry = lax.fori_loop(0, nblk // _UNROLL, body, zero)
        nt_vec[...] = carry
        pltpu.sync_copy(tid_out, tid_hbm.at[row])
        pltpu.sync_copy(st_out, st_hbm.at[row])
        pltpu.sync_copy(en_out, en_hbm.at[row])
        pltpu.sync_copy(nt_vec, xchg_hbm.at[row])

        plsc.subcore_barrier()

        @pl.when(row == 0)
        def _write_counts():
            pltpu.sync_copy(xchg_hbm, agg)
            iot = lax.iota(jnp.int32, _L)
            nt_vec[...] = plsc.load_gather(agg, [iot, iot])
            pltpu.sync_copy(nt_vec, nt_hbm)

    return tok_kernel


def kernel(chars):
    B, L = chars.shape
    tid, st, en, nt, _unused_xchg = _make_sc_kernel(B, L)(chars)
    return (tid, st, en, nt)


# speedup vs baseline: 1.0416x; 1.0137x over previous
import functools

import jax
import jax.numpy as jnp
from jax import lax
from jax.experimental import pallas as pl
from jax.experimental.pallas import tpu as pltpu
from jax.experimental.pallas import tpu_sc as plsc

_L = 16
_PAD = 128
_UNROLL = 4


def _make_sc_kernel(B, L):
    nblk = L // _L
    mesh = plsc.VectorSubcoreMesh(core_axis_name="c", subcore_axis_name="s",
                                  num_cores=1)

    @functools.partial(
        pl.kernel,
        mesh=mesh,
        compiler_params=pltpu.CompilerParams(needs_layout_passes=False),
        out_type=(
            jax.ShapeDtypeStruct((B, L), jnp.int32),
            jax.ShapeDtypeStruct((B, L), jnp.int32),
            jax.ShapeDtypeStruct((B, L), jnp.int32),
            jax.ShapeDtypeStruct((B,), jnp.int32),
            jax.ShapeDtypeStruct((B, _L), jnp.int32),
        ),
        scratch_types=(
            pltpu.VMEM((L + 2 * _PAD,), jnp.int32),
            pltpu.VMEM((L,), jnp.int32),
            pltpu.VMEM((L,), jnp.int32),
            pltpu.VMEM((L,), jnp.int32),
            pltpu.VMEM((_L,), jnp.int32),
            pltpu.VMEM((B, _L), jnp.int32),
            pltpu.SemaphoreType.DMA,
        ),
    )
    def tok_kernel(chars_hbm, tid_hbm, st_hbm, en_hbm, nt_hbm, xchg_hbm,
                   padded, tid_out, st_out, en_out, nt_vec, agg, osem):
        row = lax.axis_index("s")
        zero = jnp.zeros((_L,), jnp.int32)

        padded[pl.ds(_PAD - _L, _L)] = zero
        padded[pl.ds(_PAD + L, _L)] = zero
        pltpu.sync_copy(chars_hbm.at[row], padded.at[pl.ds(_PAD, L)])

        def body(g, cr):
            neg1 = jnp.full((_L,), -1, jnp.int32)
            one = jnp.full((_L,), 1, jnp.int32)
            gbase = g * (_L * _UNROLL)
            blocks = []
            for u in range(_UNROLL):
                base = gbase + u * _L
                st_out[pl.ds(base, _L)] = neg1
                en_out[pl.ds(base, _L)] = neg1
                c = padded[pl.ds(base + _PAD, _L)]
                p = padded[pl.ds(base + _PAD - 1, _L)]
                n = padded[pl.ds(base + _PAD + 1, _L)]
                is_tok = c != 0
                start_m = is_tok & (p == 0)
                end_m = is_tok & (n == 0)
                cum = plsc.cumsum(jnp.where(start_m, one, zero))
                pc = plsc.all_reduce_population_count(start_m)
                blocks.append((base, is_tok, start_m, end_m, cum, pc))
            for base, is_tok, start_m, end_m, cum, pc in blocks:
                tid = cr + cum - 1
                tid_out[pl.ds(base, _L)] = jnp.where(is_tok, tid, neg1)
                idx = jnp.maximum(tid, 0)
                pos = lax.iota(jnp.int32, _L) + base
                plsc.store_scatter(st_out, [idx], pos, mask=start_m)
                plsc.store_scatter(en_out, [idx], pos + 1, mask=end_m)
                cr = cr + pc
            return cr

        carry = lax.fori_loop(0, nblk // _UNROLL, body, zero)
        nt_vec[...] = carry
        cp_tid = pltpu.async_copy(tid_out, tid_hbm.at[row], osem)
        cp_st = pltpu.async_copy(st_out, st_hbm.at[row], osem)
        cp_en = pltpu.async_copy(en_out, en_hbm.at[row], osem)
        pltpu.sync_copy(nt_vec, xchg_hbm.at[row])

        plsc.subcore_barrier()

        @pl.when(row == 0)
        def _write_counts():
            pltpu.sync_copy(xchg_hbm, agg)
            iot = lax.iota(jnp.int32, _L)
            nt_vec[...] = plsc.load_gather(agg, [iot, iot])
            pltpu.sync_copy(nt_vec, nt_hbm)

        cp_tid.wait()
        cp_st.wait()
        cp_en.wait()

    return tok_kernel


def kernel(chars):
    B, L = chars.shape
    tid, st, en, nt, _unused_xchg = _make_sc_kernel(B, L)(chars)
    return (tid, st, en, nt)
